# baseline (speedup 1.0000x reference)
import jax
import jax.numpy as jnp
from jax import lax
from jax.experimental import pallas as pl
from jax.experimental.pallas import tpu as pltpu


def _make_fused_kernel(H, W, Cin, Cmid):
    HW = H * W

    def body(x_ref, w1_ref, s1_ref, b1_ref, w2_ref, b2_ref, out_ref):
        x = x_ref[0].astype(jnp.bfloat16).reshape(Cin, HW)

        y = jnp.dot(w1_ref[...], x,
                    preferred_element_type=jnp.float32)

        col = lax.broadcasted_iota(jnp.int32, (1, HW), 1)
        yy = col // W
        xx = col - yy * W
        row_ok = {-1: yy >= 1, 0: None, 1: yy <= H - 2}
        col_ok = {-1: xx >= 1, 0: None, 1: xx <= W - 2}

        acc = None
        t = 0
        for dy in (-1, 0, 1):
            for dx in (-1, 0, 1):
                s = dy * W + dx
                part = y[t * Cmid:(t + 1) * Cmid]
                if s != 0:
                    part = pltpu.roll(part, (-s) % HW, 1)
                conds = [c for c in (row_ok[dy], col_ok[dx]) if c is not None]
                if conds:
                    valid = conds[0]
                    for c in conds[1:]:
                        valid = jnp.logical_and(valid, c)
                    part = jnp.where(valid, part, 0.0)
                acc = part if acc is None else acc + part
                t += 1

        h = jnp.maximum(acc * s1_ref[...] + b1_ref[...], 0.0)

        out = jnp.dot(w2_ref[...], h, preferred_element_type=jnp.float32)
        out_ref[...] = (out + b2_ref[...])[None]

    return body


def kernel(x_nchw, w1, b1_conv, gamma, beta, mean, var, eps, w2, b2):
    N, Cin, H, W = x_nchw.shape
    Cmid = w1.shape[0]
    Cout = w2.shape[0]
    HW = H * W

    w1_k = (jnp.transpose(w1, (2, 3, 0, 1))
            .reshape(9 * Cmid, Cin).astype(jnp.bfloat16))

    scale = gamma / jnp.sqrt(var + eps)
    bias = (b1_conv - mean) * scale + beta
    s1 = scale.reshape(Cmid, 1).astype(jnp.float32)
    b1 = bias.reshape(Cmid, 1).astype(jnp.float32)

    w2_k = w2[:, :, 0, 0].astype(jnp.float32)
    b2_k = b2.reshape(Cout, 1).astype(jnp.float32)

    out_flat = pl.pallas_call(
        _make_fused_kernel(H, W, Cin, Cmid),
        out_shape=jax.ShapeDtypeStruct((N, Cout, HW), jnp.float32),
        grid=(N,),
        in_specs=[
            pl.BlockSpec((1, Cin, H, W), lambda n: (n, 0, 0, 0)),
            pl.BlockSpec((9 * Cmid, Cin), lambda n: (0, 0)),
            pl.BlockSpec((Cmid, 1), lambda n: (0, 0)),
            pl.BlockSpec((Cmid, 1), lambda n: (0, 0)),
            pl.BlockSpec((Cout, Cmid), lambda n: (0, 0)),
            pl.BlockSpec((Cout, 1), lambda n: (0, 0)),
        ],
        out_specs=pl.BlockSpec((1, Cout, HW), lambda n: (n, 0, 0)),
        compiler_params=pltpu.CompilerParams(
            dimension_semantics=("parallel",),
            vmem_limit_bytes=64 * 1024 * 1024),
    )(x_nchw, w1_k, s1, b1, w2_k, b2_k)

    return out_flat.reshape(N, Cout, H, W)


# speedup vs baseline: 1.5293x; 1.5293x over previous
import jax
import jax.numpy as jnp
from jax import lax
from jax.experimental import pallas as pl
from jax.experimental.pallas import tpu as pltpu


def _make_fused_kernel(H, W, Cin, Cmid):
    HW = H * W

    def body(x_ref, w1_ref, s1_ref, b1_ref, w2_ref, b2_ref, out_ref):
        x = x_ref[0].astype(jnp.bfloat16).reshape(Cin, HW)

        y = jnp.dot(w1_ref[...], x,
                    preferred_element_type=jnp.float32)

        col = lax.broadcasted_iota(jnp.int32, (1, HW), 1)
        yy = col // W
        xx = col - yy * W
        row_ok = {-1: yy >= 1, 0: None, 1: yy <= H - 2}
        col_ok = {-1: xx >= 1, 0: None, 1: xx <= W - 2}

        acc = None
        t = 0
        for dy in (-1, 0, 1):
            for dx in (-1, 0, 1):
                s = dy * W + dx
                part = y[t * Cmid:(t + 1) * Cmid]
                if s != 0:
                    part = pltpu.roll(part, (-s) % HW, 1)
                conds = [c for c in (row_ok[dy], col_ok[dx]) if c is not None]
                if conds:
                    valid = conds[0]
                    for c in conds[1:]:
                        valid = jnp.logical_and(valid, c)
                    part = jnp.where(valid, part, 0.0)
                acc = part if acc is None else acc + part
                t += 1

        h = jnp.maximum(acc * s1_ref[...] + b1_ref[...], 0.0)

        out = jnp.dot(w2_ref[...], h, preferred_element_type=jnp.float32)
        out_ref[...] = (out + b2_ref[...])[None]

    return body


def kernel(x_nchw, w1, b1_conv, gamma, beta, mean, var, eps, w2, b2):
    N, Cin, H, W = x_nchw.shape
    Cmid = w1.shape[0]
    Cout = w2.shape[0]
    HW = H * W

    x_m = x_nchw.reshape(N, Cin, HW // 128, 128)

    w1_k = (jnp.transpose(w1, (2, 3, 0, 1))
            .reshape(9 * Cmid, Cin).astype(jnp.bfloat16))

    scale = gamma / jnp.sqrt(var + eps)
    bias = (b1_conv - mean) * scale + beta
    s1 = scale.reshape(Cmid, 1).astype(jnp.float32)
    b1 = bias.reshape(Cmid, 1).astype(jnp.float32)

    w2_k = w2[:, :, 0, 0].astype(jnp.float32)
    b2_k = b2.reshape(Cout, 1).astype(jnp.float32)

    out_flat = pl.pallas_call(
        _make_fused_kernel(H, W, Cin, Cmid),
        out_shape=jax.ShapeDtypeStruct((N, Cout, HW), jnp.float32),
        grid=(N,),
        in_specs=[
            pl.BlockSpec((1, Cin, HW // 128, 128), lambda n: (n, 0, 0, 0)),
            pl.BlockSpec((9 * Cmid, Cin), lambda n: (0, 0)),
            pl.BlockSpec((Cmid, 1), lambda n: (0, 0)),
            pl.BlockSpec((Cmid, 1), lambda n: (0, 0)),
            pl.BlockSpec((Cout, Cmid), lambda n: (0, 0)),
            pl.BlockSpec((Cout, 1), lambda n: (0, 0)),
        ],
        out_specs=pl.BlockSpec((1, Cout, HW), lambda n: (n, 0, 0)),
        compiler_params=pltpu.CompilerParams(
            dimension_semantics=("parallel",),
            vmem_limit_bytes=64 * 1024 * 1024),
    )(x_m, w1_k, s1, b1, w2_k, b2_k)

    return out_flat.reshape(N, Cout, H, W)


# speedup vs baseline: 2.9982x; 1.9605x over previous
import jax
import jax.numpy as jnp
from jax import lax
from jax.experimental import pallas as pl
from jax.experimental.pallas import tpu as pltpu


def _make_fused_kernel(H, W, Cin, Cmid):
    HW = H * W

    def body(x_ref, w1_ref, s1_ref, b1_ref, w2_ref, b2_ref, out_ref):
        x = x_ref[0].astype(jnp.bfloat16)

        y = lax.dot_general(w1_ref[...], x,
                            (((1,), (1,)), ((), ())),
                            preferred_element_type=jnp.float32)

        col = lax.broadcasted_iota(jnp.int32, (1, HW), 1)
        yy = col // W
        xx = col - yy * W
        row_ok = {-1: yy >= 1, 0: None, 1: yy <= H - 2}
        col_ok = {-1: xx >= 1, 0: None, 1: xx <= W - 2}

        acc = None
        t = 0
        for dy in (-1, 0, 1):
            for dx in (-1, 0, 1):
                s = dy * W + dx
                part = y[t * Cmid:(t + 1) * Cmid]
                if s != 0:
                    part = pltpu.roll(part, (-s) % HW, 1)
                conds = [c for c in (row_ok[dy], col_ok[dx]) if c is not None]
                if conds:
                    valid = conds[0]
                    for c in conds[1:]:
                        valid = jnp.logical_and(valid, c)
                    part = jnp.where(valid, part, 0.0)
                acc = part if acc is None else acc + part
                t += 1

        h = jnp.maximum(acc * s1_ref[...] + b1_ref[...], 0.0)

        out = jnp.dot(w2_ref[...], h, preferred_element_type=jnp.float32)
        out_ref[...] = (out + b2_ref[...])[None]

    return body


def kernel(x_nchw, w1, b1_conv, gamma, beta, mean, var, eps, w2, b2):
    N, Cin, H, W = x_nchw.shape
    Cmid = w1.shape[0]
    Cout = w2.shape[0]
    HW = H * W

    x_t = jnp.transpose(x_nchw, (0, 2, 3, 1)).reshape(N, HW, Cin)

    w1_k = (jnp.transpose(w1, (2, 3, 0, 1))
            .reshape(9 * Cmid, Cin).astype(jnp.bfloat16))

    scale = gamma / jnp.sqrt(var + eps)
    bias = (b1_conv - mean) * scale + beta
    s1 = scale.reshape(Cmid, 1).astype(jnp.float32)
    b1 = bias.reshape(Cmid, 1).astype(jnp.float32)

    w2_k = w2[:, :, 0, 0].astype(jnp.float32)
    b2_k = b2.reshape(Cout, 1).astype(jnp.float32)

    out_flat = pl.pallas_call(
        _make_fused_kernel(H, W, Cin, Cmid),
        out_shape=jax.ShapeDtypeStruct((N, Cout, HW), jnp.float32),
        grid=(N,),
        in_specs=[
            pl.BlockSpec((1, HW, Cin), lambda n: (n, 0, 0)),
            pl.BlockSpec((9 * Cmid, Cin), lambda n: (0, 0)),
            pl.BlockSpec((Cmid, 1), lambda n: (0, 0)),
            pl.BlockSpec((Cmid, 1), lambda n: (0, 0)),
            pl.BlockSpec((Cout, Cmid), lambda n: (0, 0)),
            pl.BlockSpec((Cout, 1), lambda n: (0, 0)),
        ],
        out_specs=pl.BlockSpec((1, Cout, HW), lambda n: (n, 0, 0)),
        compiler_params=pltpu.CompilerParams(
            dimension_semantics=("parallel",),
            vmem_limit_bytes=64 * 1024 * 1024),
    )(x_t, w1_k, s1, b1, w2_k, b2_k)

    return out_flat.reshape(N, Cout, H, W)
